# baseline (speedup 1.0000x reference)
import functools

import jax
import jax.numpy as jnp
from jax import lax
from jax.experimental import pallas as pl
from jax.experimental.pallas import tpu as pltpu
from jax.experimental.pallas import tpu_sc as plsc

DIM = 64
BATCH = 4096
NUM_NEG = 8
NUM_CORES = 2
NUM_SUBCORES = 16
NUM_WORKERS = NUM_CORES * NUM_SUBCORES
BPW = BATCH // NUM_WORKERS
GROUPS = BPW // 16


def _sc_body(users_h, items_h, negs_h, uemb_h, iemb_h, pos_h, nexp_h,
             u_idx, i_idx, n_idx, u_rows, i_rows, n_rows, pos_v, nexp_v,
             sem):
  wid = lax.axis_index("s") * NUM_CORES + lax.axis_index("c")
  base = wid * BPW

  pltpu.sync_copy(users_h.at[pl.ds(base, BPW)], u_idx)
  pltpu.sync_copy(items_h.at[pl.ds(base, BPW)], i_idx)
  for k in range(NUM_NEG):
    pltpu.sync_copy(negs_h.at[pl.ds(k * BATCH + base, BPW)], n_idx.at[k])

  copies = [
      pltpu.async_copy(uemb_h.at[u_idx], u_rows, sem),
      pltpu.async_copy(iemb_h.at[i_idx], i_rows, sem),
  ]
  for k in range(NUM_NEG):
    copies.append(pltpu.async_copy(iemb_h.at[n_idx.at[k]], n_rows.at[k], sem))
  for cp in copies:
    cp.wait()

  iota = lax.iota(jnp.int32, 16)
  zero = jnp.zeros((16,), jnp.float32)
  for g in range(GROUPS):
    row = iota + (16 * g)

    def dim_body(d, carry, row=row):
      col = jnp.full((16,), d, jnp.int32)
      u_d = plsc.load_gather(u_rows, [row, col])
      p = carry[0] + u_d * plsc.load_gather(i_rows, [row, col])
      ns = []
      for k in range(NUM_NEG):
        kk = jnp.full((16,), k, jnp.int32)
        ns.append(carry[1 + k] + u_d * plsc.load_gather(n_rows, [kk, row, col]))
      return (p, *ns)

    scores = lax.fori_loop(0, DIM, dim_body, (zero,) * (1 + NUM_NEG))
    pos_v[pl.ds(16 * g, 16)] = scores[0]
    nexp = jnp.exp(scores[1])
    for k in range(1, NUM_NEG):
      nexp = nexp + jnp.exp(scores[1 + k])
    nexp_v[pl.ds(16 * g, 16)] = nexp

  pltpu.sync_copy(pos_v, pos_h.at[pl.ds(base, BPW)])
  pltpu.sync_copy(nexp_v, nexp_h.at[pl.ds(base, BPW)])


_sc_scores = functools.partial(
    pl.kernel,
    mesh=plsc.VectorSubcoreMesh(core_axis_name="c", subcore_axis_name="s"),
    out_type=[
        jax.ShapeDtypeStruct((BATCH,), jnp.float32),
        jax.ShapeDtypeStruct((BATCH,), jnp.float32),
    ],
    scratch_types=[
        pltpu.VMEM((BPW,), jnp.int32),
        pltpu.VMEM((BPW,), jnp.int32),
        pltpu.VMEM((NUM_NEG, BPW), jnp.int32),
        pltpu.VMEM((BPW, DIM), jnp.float32),
        pltpu.VMEM((BPW, DIM), jnp.float32),
        pltpu.VMEM((NUM_NEG, BPW, DIM), jnp.float32),
        pltpu.VMEM((BPW,), jnp.float32),
        pltpu.VMEM((BPW,), jnp.float32),
        pltpu.SemaphoreType.DMA,
    ],
    compiler_params=pltpu.CompilerParams(
        needs_layout_passes=False, use_tc_tiling_on_sc=False),
)(_sc_body)


def _tc_loss_body(pos_ref, nexp_ref, o_ref):
  pe = jnp.exp(pos_ref[...])
  ne = nexp_ref[...]
  losses = -jnp.log(pe / (pe + ne))
  o_ref[0, 0] = jnp.sum(losses) * (1.0 / BATCH)


_tc_loss = pl.pallas_call(
    _tc_loss_body,
    out_shape=jax.ShapeDtypeStruct((1, 1), jnp.float32),
    out_specs=pl.BlockSpec(memory_space=pltpu.SMEM),
)


def kernel(users, items, negatives, user_emb, item_emb):
  users = users.astype(jnp.int32)
  items = items.astype(jnp.int32)
  negatives = negatives.astype(jnp.int32)
  pos, nexp = _sc_scores(users, items, negatives, user_emb, item_emb)
  out = _tc_loss(pos.reshape(32, 128), nexp.reshape(32, 128))
  return out[0, 0]
